# baseline (speedup 1.0000x reference)
import functools

import jax
import jax.numpy as jnp
from jax import lax
from jax.experimental import pallas as pl
from jax.experimental.pallas import tpu as pltpu
from jax.experimental.pallas import tpu_sc as plsc

N = 10000
E = 320000
NP = 10240
BR = 640
G = NP // BR
NTILES = 32
EPT = E // NTILES
CH = 128
NCH = 82
EPTP = NCH * CH
DUMMY = N


@functools.lru_cache(maxsize=None)
def _mesh():
  return plsc.VectorSubcoreMesh(core_axis_name="c", subcore_axis_name="s",
                                num_cores=2, num_subcores=16)


@functools.lru_cache(maxsize=None)
def _make_sc_scatter(F):
  def body(src_hbm, dst_hbm, h_hbm, zeros_hbm, out_hbm,
           sidx_v, didx_v, rows_v, acc_sh, gsem):
    core = lax.axis_index("c")
    sub = lax.axis_index("s")
    wid = sub * 2 + core
    pltpu.sync_copy(zeros_hbm, acc_sh.at[pl.ds(sub * BR, BR)])
    pltpu.sync_copy(src_hbm.at[wid], sidx_v)
    pltpu.sync_copy(dst_hbm.at[wid], didx_v)
    plsc.subcore_barrier()

    def chunk(c, carry):
      pltpu.async_copy(h_hbm.at[sidx_v.at[c]], rows_v, gsem).wait()
      pltpu.sync_copy(rows_v, acc_sh.at[didx_v.at[c]], add=True)
      return carry

    lax.fori_loop(0, NCH, chunk, 0)
    plsc.subcore_barrier()
    pltpu.sync_copy(acc_sh.at[pl.ds(sub * BR, BR)],
                    out_hbm.at[core, pl.ds(sub * BR, BR)])

  return pl.kernel(
      body,
      out_type=jax.ShapeDtypeStruct((2, NP, F), jnp.float32),
      mesh=_mesh(),
      scratch_types=[
          pltpu.VMEM((NCH, CH), jnp.int32),
          pltpu.VMEM((NCH, CH), jnp.int32),
          pltpu.VMEM((CH, F), jnp.float32),
          pltpu.VMEM_SHARED((NP, F), jnp.float32),
          pltpu.SemaphoreType.DMA,
      ],
      compiler_params=pltpu.CompilerParams(use_tc_tiling_on_sc=False),
  )


@functools.lru_cache(maxsize=None)
def _make_sc_deg():
  def body(dst_hbm, ones_hbm, zeros_hbm, out_hbm, didx_v, ones_v, acc_sh):
    core = lax.axis_index("c")
    sub = lax.axis_index("s")
    wid = sub * 2 + core
    pltpu.sync_copy(zeros_hbm, acc_sh.at[pl.ds(sub * BR, BR)])
    pltpu.sync_copy(dst_hbm.at[wid], didx_v)
    pltpu.sync_copy(ones_hbm, ones_v)
    plsc.subcore_barrier()

    def chunk(c, carry):
      pltpu.sync_copy(ones_v, acc_sh.at[didx_v.at[c]], add=True)
      return carry

    lax.fori_loop(0, NCH, chunk, 0)
    plsc.subcore_barrier()
    pltpu.sync_copy(acc_sh.at[pl.ds(sub * BR, BR)],
                    out_hbm.at[core, pl.ds(sub * BR, BR)])

  return pl.kernel(
      body,
      out_type=jax.ShapeDtypeStruct((2, NP, 16), jnp.float32),
      mesh=_mesh(),
      scratch_types=[
          pltpu.VMEM((NCH, CH), jnp.int32),
          pltpu.VMEM((CH, 16), jnp.float32),
          pltpu.VMEM_SHARED((NP, 16), jnp.float32),
      ],
      compiler_params=pltpu.CompilerParams(use_tc_tiling_on_sc=False),
  )


def _dis_block(degp_ref):
  d = degp_ref[0, :, 0:1] + degp_ref[1, :, 0:1] + 1.0
  return lax.rsqrt(d)


def _mm1_body(degp_ref, x_ref, w_ref, o_ref):
  dis = _dis_block(degp_ref)
  h = jnp.dot(x_ref[...], w_ref[...], preferred_element_type=jnp.float32)
  o_ref[...] = h * dis


def _mm2_body(degp_ref, p_ref, h1s_ref, b1_ref, w_ref, o_ref):
  dis = _dis_block(degp_ref)
  agg = p_ref[0] + p_ref[1] + h1s_ref[...]
  z = jnp.maximum(agg * dis + b1_ref[...][None, :], 0.0)
  h2 = jnp.dot(z, w_ref[...], preferred_element_type=jnp.float32)
  o_ref[...] = h2 * dis


def _out_body(degp_ref, q_ref, h2s_ref, b2_ref, o_ref):
  dis = _dis_block(degp_ref)
  o = (q_ref[0] + q_ref[1] + h2s_ref[...]) * dis + b2_ref[...][None, :]
  m = jnp.max(o, axis=1, keepdims=True)
  e = jnp.exp(o - m)
  s = jnp.sum(e, axis=1, keepdims=True)
  o_ref[...] = (o - m) - jnp.log(s)


def _degp_spec():
  return pl.BlockSpec((2, BR, 16), lambda i: (0, i, 0))


_mm1 = pl.pallas_call(
    _mm1_body,
    grid=(G,),
    in_specs=[
        _degp_spec(),
        pl.BlockSpec((BR, 128), lambda i: (i, 0)),
        pl.BlockSpec((128, 128), lambda i: (0, 0)),
    ],
    out_specs=pl.BlockSpec((BR, 128), lambda i: (i, 0)),
    out_shape=jax.ShapeDtypeStruct((NP, 128), jnp.float32),
)

_mm2 = pl.pallas_call(
    _mm2_body,
    grid=(G,),
    in_specs=[
        _degp_spec(),
        pl.BlockSpec((2, BR, 128), lambda i: (0, i, 0)),
        pl.BlockSpec((BR, 128), lambda i: (i, 0)),
        pl.BlockSpec((128,), lambda i: (0,)),
        pl.BlockSpec((128, 16), lambda i: (0, 0)),
    ],
    out_specs=pl.BlockSpec((BR, 16), lambda i: (i, 0)),
    out_shape=jax.ShapeDtypeStruct((NP, 16), jnp.float32),
)

_outk = pl.pallas_call(
    _out_body,
    grid=(G,),
    in_specs=[
        _degp_spec(),
        pl.BlockSpec((2, BR, 16), lambda i: (0, i, 0)),
        pl.BlockSpec((BR, 16), lambda i: (i, 0)),
        pl.BlockSpec((16,), lambda i: (0,)),
    ],
    out_specs=pl.BlockSpec((BR, 16), lambda i: (i, 0)),
    out_shape=jax.ShapeDtypeStruct((N, 16), jnp.float32),
)


@jax.jit
def kernel(x, edge_index, W1, b1, W2, b2):
  src = edge_index[0].reshape(NTILES, EPT)
  dst = edge_index[1].reshape(NTILES, EPT)
  pad = EPTP - EPT
  src_p = jnp.pad(src, ((0, 0), (0, pad))).reshape(NTILES, NCH, CH)
  pad_dst = DUMMY + (jnp.arange(pad, dtype=jnp.int32)[None, :]
                     + 29 * jnp.arange(NTILES, dtype=jnp.int32)[:, None]
                     ) % (NP - N)
  dst_p = jnp.concatenate([dst, pad_dst], axis=1).reshape(NTILES, NCH, CH)
  zeros128 = jnp.zeros((BR, 128), jnp.float32)
  zeros16 = jnp.zeros((BR, 16), jnp.float32)
  ones16 = jnp.ones((CH, 16), jnp.float32)

  degp = _make_sc_deg()(dst_p, ones16, zeros16)
  h1s = _mm1(degp, x, W1)
  p = _make_sc_scatter(128)(src_p, dst_p, h1s, zeros128)
  h2s = _mm2(degp, p, h1s, b1, W2)
  q = _make_sc_scatter(16)(src_p, dst_p, h2s, zeros16)
  return _outk(degp, q, h2s, b2)


# speedup vs baseline: 2.6693x; 2.6693x over previous
import functools

import jax
import jax.numpy as jnp
from jax import lax
from jax.experimental import pallas as pl
from jax.experimental.pallas import tpu as pltpu
from jax.experimental.pallas import tpu_sc as plsc

N = 10000
E = 320000
NP = 10240
BR = 640
G = NP // BR
NTILES = 32
EPT = E // NTILES
CH = 128
NCH = 82
EPTP = NCH * CH
DUMMY = N


@functools.lru_cache(maxsize=None)
def _mesh():
  return plsc.VectorSubcoreMesh(core_axis_name="c", subcore_axis_name="s",
                                num_cores=2, num_subcores=16)


@functools.lru_cache(maxsize=None)
def _make_sc_scatter(F):
  def body(src_hbm, dst_hbm, h_hbm, zeros_hbm, out_hbm,
           sidx_v, didx_v, rows_v, acc_sh, gsem):
    core = lax.axis_index("c")
    sub = lax.axis_index("s")
    wid = sub * 2 + core
    pltpu.sync_copy(zeros_hbm, acc_sh.at[pl.ds(sub * BR, BR)])
    pltpu.sync_copy(src_hbm.at[wid], sidx_v)
    pltpu.sync_copy(dst_hbm.at[wid], didx_v)
    plsc.subcore_barrier()

    def chunk(c, carry):
      pltpu.async_copy(h_hbm.at[sidx_v.at[c]], rows_v, gsem).wait()
      pltpu.sync_copy(rows_v, acc_sh.at[didx_v.at[c]], add=True)
      return carry

    lax.fori_loop(0, NCH, chunk, 0)
    plsc.subcore_barrier()
    pltpu.sync_copy(acc_sh.at[pl.ds(sub * BR, BR)],
                    out_hbm.at[core, pl.ds(sub * BR, BR)])

  return pl.kernel(
      body,
      out_type=jax.ShapeDtypeStruct((2, NP, F), jnp.float32),
      mesh=_mesh(),
      scratch_types=[
          pltpu.VMEM((NCH, CH), jnp.int32),
          pltpu.VMEM((NCH, CH), jnp.int32),
          pltpu.VMEM((CH, F), jnp.float32),
          pltpu.VMEM_SHARED((NP, F), jnp.float32),
          pltpu.SemaphoreType.DMA,
      ],
      compiler_params=pltpu.CompilerParams(use_tc_tiling_on_sc=False),
  )


@functools.lru_cache(maxsize=None)
def _make_sc_deg():
  def body(dst_hbm, ones_hbm, zeros_hbm, out_hbm, didx_v, ones_v, acc_sh):
    core = lax.axis_index("c")
    sub = lax.axis_index("s")
    wid = sub * 2 + core
    pltpu.sync_copy(zeros_hbm, acc_sh.at[pl.ds(sub * BR, BR)])
    pltpu.sync_copy(dst_hbm.at[wid], didx_v)
    pltpu.sync_copy(ones_hbm, ones_v)
    plsc.subcore_barrier()

    def chunk(c, carry):
      pltpu.sync_copy(ones_v, acc_sh.at[didx_v.at[c]], add=True)
      return carry

    lax.fori_loop(0, NCH, chunk, 0)
    plsc.subcore_barrier()
    pltpu.sync_copy(acc_sh.at[pl.ds(sub * BR, BR)],
                    out_hbm.at[core, pl.ds(sub * BR, BR)])

  return pl.kernel(
      body,
      out_type=jax.ShapeDtypeStruct((2, NP, 16), jnp.float32),
      mesh=_mesh(),
      scratch_types=[
          pltpu.VMEM((NCH, CH), jnp.int32),
          pltpu.VMEM((CH, 16), jnp.float32),
          pltpu.VMEM_SHARED((NP, 16), jnp.float32),
      ],
      compiler_params=pltpu.CompilerParams(use_tc_tiling_on_sc=False),
  )


def _dis_block(degp_ref):
  d = degp_ref[0, :, 0:1] + degp_ref[1, :, 0:1] + 1.0
  return lax.rsqrt(d)


def _mm1_body(degp_ref, x_ref, w_ref, o_ref):
  dis = _dis_block(degp_ref)
  h = jnp.dot(x_ref[...], w_ref[...], preferred_element_type=jnp.float32)
  o_ref[...] = h * dis


def _mm2_body(degp_ref, p_ref, h1s_ref, b1_ref, w_ref, o_ref):
  dis = _dis_block(degp_ref)
  agg = p_ref[0] + p_ref[1] + h1s_ref[...]
  z = jnp.maximum(agg * dis + b1_ref[...][None, :], 0.0)
  h2 = jnp.dot(z, w_ref[...], preferred_element_type=jnp.float32)
  o_ref[...] = h2 * dis


def _out_body(degp_ref, q_ref, h2s_ref, b2_ref, o_ref):
  dis = _dis_block(degp_ref)
  o = (q_ref[0] + q_ref[1] + h2s_ref[...]) * dis + b2_ref[...][None, :]
  m = jnp.max(o, axis=1, keepdims=True)
  e = jnp.exp(o - m)
  s = jnp.sum(e, axis=1, keepdims=True)
  o_ref[...] = (o - m) - jnp.log(s)


def _degp_spec():
  return pl.BlockSpec((2, BR, 16), lambda i: (0, i, 0))


_mm1 = pl.pallas_call(
    _mm1_body,
    grid=(G,),
    in_specs=[
        _degp_spec(),
        pl.BlockSpec((BR, 128), lambda i: (i, 0)),
        pl.BlockSpec((128, 128), lambda i: (0, 0)),
    ],
    out_specs=pl.BlockSpec((BR, 128), lambda i: (i, 0)),
    out_shape=jax.ShapeDtypeStruct((NP, 128), jnp.float32),
)

_mm2 = pl.pallas_call(
    _mm2_body,
    grid=(G,),
    in_specs=[
        _degp_spec(),
        pl.BlockSpec((2, BR, 128), lambda i: (0, i, 0)),
        pl.BlockSpec((BR, 128), lambda i: (i, 0)),
        pl.BlockSpec((128,), lambda i: (0,)),
        pl.BlockSpec((128, 16), lambda i: (0, 0)),
    ],
    out_specs=pl.BlockSpec((BR, 16), lambda i: (i, 0)),
    out_shape=jax.ShapeDtypeStruct((NP, 16), jnp.float32),
)

_outk = pl.pallas_call(
    _out_body,
    grid=(G,),
    in_specs=[
        _degp_spec(),
        pl.BlockSpec((2, BR, 16), lambda i: (0, i, 0)),
        pl.BlockSpec((BR, 16), lambda i: (i, 0)),
        pl.BlockSpec((16,), lambda i: (0,)),
    ],
    out_specs=pl.BlockSpec((BR, 16), lambda i: (i, 0)),
    out_shape=jax.ShapeDtypeStruct((N, 16), jnp.float32),
)


@jax.jit
def kernel(x, edge_index, W1, b1, W2, b2):
  src = edge_index[0].reshape(NTILES, EPT)
  dst = edge_index[1].reshape(NTILES, EPT)
  pad = EPTP - EPT
  pad_rows = DUMMY + (jnp.arange(pad, dtype=jnp.int32)[None, :]
                      + 29 * jnp.arange(NTILES, dtype=jnp.int32)[:, None]
                      ) % (NP - N)
  src_p = jnp.concatenate([src, pad_rows], axis=1).reshape(NTILES, NCH, CH)
  dst_p = jnp.concatenate([dst, pad_rows], axis=1).reshape(NTILES, NCH, CH)
  zeros128 = jnp.zeros((BR, 128), jnp.float32)
  zeros16 = jnp.zeros((BR, 16), jnp.float32)
  ones16 = jnp.ones((CH, 16), jnp.float32)

  degp = _make_sc_deg()(dst_p, ones16, zeros16)
  h1s = _mm1(degp, x, W1)
  p = _make_sc_scatter(128)(src_p, dst_p, h1s, zeros128)
  h2s = _mm2(degp, p, h1s, b1, W2)
  q = _make_sc_scatter(16)(src_p, dst_p, h2s, zeros16)
  return _outk(degp, q, h2s, b2)


# speedup vs baseline: 2.7106x; 1.0155x over previous
import functools

import jax
import jax.numpy as jnp
from jax import lax
from jax.experimental import pallas as pl
from jax.experimental.pallas import tpu as pltpu
from jax.experimental.pallas import tpu_sc as plsc

N = 10000
E = 320000
NP = 10240
BR = 640
G = NP // BR
NTILES = 32
EPT = E // NTILES
CH = 128
NCH = 80
EPTP = NCH * CH
DUMMY = N


@functools.lru_cache(maxsize=None)
def _mesh():
  return plsc.VectorSubcoreMesh(core_axis_name="c", subcore_axis_name="s",
                                num_cores=2, num_subcores=16)


@functools.lru_cache(maxsize=None)
def _make_sc_scatter(F):
  def body(src_hbm, dst_hbm, h_hbm, zeros_hbm, out_hbm,
           sidx_v, didx_v, rows_v, acc_sh, gsem):
    core = lax.axis_index("c")
    sub = lax.axis_index("s")
    wid = sub * 2 + core
    pltpu.sync_copy(zeros_hbm, acc_sh.at[pl.ds(sub * BR, BR)])
    pltpu.sync_copy(src_hbm.at[wid], sidx_v)
    pltpu.sync_copy(dst_hbm.at[wid], didx_v)
    plsc.subcore_barrier()

    def chunk(c, carry):
      pltpu.async_copy(h_hbm.at[sidx_v.at[c]], rows_v, gsem).wait()
      pltpu.sync_copy(rows_v, acc_sh.at[didx_v.at[c]], add=True)
      return carry

    lax.fori_loop(0, NCH, chunk, 0)
    plsc.subcore_barrier()
    pltpu.sync_copy(acc_sh.at[pl.ds(sub * BR, BR)],
                    out_hbm.at[core, pl.ds(sub * BR, BR)])

  return pl.kernel(
      body,
      out_type=jax.ShapeDtypeStruct((2, NP, F), jnp.float32),
      mesh=_mesh(),
      scratch_types=[
          pltpu.VMEM((NCH, CH), jnp.int32),
          pltpu.VMEM((NCH, CH), jnp.int32),
          pltpu.VMEM((CH, F), jnp.float32),
          pltpu.VMEM_SHARED((NP, F), jnp.float32),
          pltpu.SemaphoreType.DMA,
      ],
      compiler_params=pltpu.CompilerParams(use_tc_tiling_on_sc=False),
  )


@functools.lru_cache(maxsize=None)
def _make_sc_deg():
  def body(dst_hbm, ones_hbm, zeros_hbm, out_hbm, didx_v, ones_v, acc_sh):
    core = lax.axis_index("c")
    sub = lax.axis_index("s")
    wid = sub * 2 + core
    pltpu.sync_copy(zeros_hbm, acc_sh.at[pl.ds(sub * BR, BR)])
    pltpu.sync_copy(dst_hbm.at[wid], didx_v)
    pltpu.sync_copy(ones_hbm, ones_v)
    plsc.subcore_barrier()

    def chunk(c, carry):
      pltpu.sync_copy(ones_v, acc_sh.at[didx_v.at[c]], add=True)
      return carry

    lax.fori_loop(0, NCH, chunk, 0)
    plsc.subcore_barrier()
    pltpu.sync_copy(acc_sh.at[pl.ds(sub * BR, BR)],
                    out_hbm.at[core, pl.ds(sub * BR, BR)])

  return pl.kernel(
      body,
      out_type=jax.ShapeDtypeStruct((2, NP, 16), jnp.float32),
      mesh=_mesh(),
      scratch_types=[
          pltpu.VMEM((NCH, CH), jnp.int32),
          pltpu.VMEM((CH, 16), jnp.float32),
          pltpu.VMEM_SHARED((NP, 16), jnp.float32),
      ],
      compiler_params=pltpu.CompilerParams(use_tc_tiling_on_sc=False),
  )


def _dis_block(degp_ref):
  d = degp_ref[0, :, 0:1] + degp_ref[1, :, 0:1] + 1.0
  return lax.rsqrt(d)


def _mm1_body(degp_ref, x_ref, w_ref, o_ref):
  dis = _dis_block(degp_ref)
  h = jnp.dot(x_ref[...], w_ref[...], preferred_element_type=jnp.float32)
  o_ref[...] = h * dis


def _mm2_body(degp_ref, p_ref, h1s_ref, b1_ref, w_ref, o_ref):
  dis = _dis_block(degp_ref)
  agg = p_ref[0] + p_ref[1] + h1s_ref[...]
  z = jnp.maximum(agg * dis + b1_ref[...][None, :], 0.0)
  h2 = jnp.dot(z, w_ref[...], preferred_element_type=jnp.float32)
  o_ref[...] = h2 * dis


def _out_body(degp_ref, q_ref, h2s_ref, b2_ref, o_ref):
  dis = _dis_block(degp_ref)
  o = (q_ref[0] + q_ref[1] + h2s_ref[...]) * dis + b2_ref[...][None, :]
  m = jnp.max(o, axis=1, keepdims=True)
  e = jnp.exp(o - m)
  s = jnp.sum(e, axis=1, keepdims=True)
  o_ref[...] = (o - m) - jnp.log(s)


def _degp_spec():
  return pl.BlockSpec((2, BR, 16), lambda i: (0, i, 0))


_mm1 = pl.pallas_call(
    _mm1_body,
    grid=(G,),
    in_specs=[
        _degp_spec(),
        pl.BlockSpec((BR, 128), lambda i: (i, 0)),
        pl.BlockSpec((128, 128), lambda i: (0, 0)),
    ],
    out_specs=pl.BlockSpec((BR, 128), lambda i: (i, 0)),
    out_shape=jax.ShapeDtypeStruct((NP, 128), jnp.float32),
)

_mm2 = pl.pallas_call(
    _mm2_body,
    grid=(G,),
    in_specs=[
        _degp_spec(),
        pl.BlockSpec((2, BR, 128), lambda i: (0, i, 0)),
        pl.BlockSpec((BR, 128), lambda i: (i, 0)),
        pl.BlockSpec((128,), lambda i: (0,)),
        pl.BlockSpec((128, 16), lambda i: (0, 0)),
    ],
    out_specs=pl.BlockSpec((BR, 16), lambda i: (i, 0)),
    out_shape=jax.ShapeDtypeStruct((NP, 16), jnp.float32),
)

_outk = pl.pallas_call(
    _out_body,
    grid=(G,),
    in_specs=[
        _degp_spec(),
        pl.BlockSpec((2, BR, 16), lambda i: (0, i, 0)),
        pl.BlockSpec((BR, 16), lambda i: (i, 0)),
        pl.BlockSpec((16,), lambda i: (0,)),
    ],
    out_specs=pl.BlockSpec((BR, 16), lambda i: (i, 0)),
    out_shape=jax.ShapeDtypeStruct((N, 16), jnp.float32),
)


@jax.jit
def kernel(x, edge_index, W1, b1, W2, b2):
  src = edge_index[0].reshape(NTILES, EPT)
  dst = edge_index[1].reshape(NTILES, EPT)
  pad = EPTP - EPT
  pad_rows = DUMMY + (jnp.arange(pad, dtype=jnp.int32)[None, :]
                      + 29 * jnp.arange(NTILES, dtype=jnp.int32)[:, None]
                      ) % (NP - N)
  src_p = jnp.concatenate([src, pad_rows], axis=1).reshape(NTILES, NCH, CH)
  dst_p = jnp.concatenate([dst, pad_rows], axis=1).reshape(NTILES, NCH, CH)
  zeros128 = jnp.zeros((BR, 128), jnp.float32)
  zeros16 = jnp.zeros((BR, 16), jnp.float32)
  ones16 = jnp.ones((CH, 16), jnp.float32)

  degp = _make_sc_deg()(dst_p, ones16, zeros16)
  h1s = _mm1(degp, x, W1)
  p = _make_sc_scatter(128)(src_p, dst_p, h1s, zeros128)
  h2s = _mm2(degp, p, h1s, b1, W2)
  q = _make_sc_scatter(16)(src_p, dst_p, h2s, zeros16)
  return _outk(degp, q, h2s, b2)
